# baseline (speedup 1.0000x reference)
import functools

import jax
import jax.numpy as jnp
from jax import lax
from jax.experimental import pallas as pl
from jax.experimental.pallas import tpu as pltpu
from jax.experimental.pallas import tpu_sc as plsc

B = 16384
EMB = 32
OUT_D = EMB + 1
NC, NS, L = 2, 16, 16
NW = NC * NS
BW = B // NW
CHUNK = 128
NCH = BW // CHUNK

_mesh = plsc.VectorSubcoreMesh(core_axis_name="c", subcore_axis_name="s")


def _assemble_chunk(idx_ref, rows_ref, feat_ref, j):
  lanes = lax.iota(jnp.int32, L)
  zeros = jnp.zeros((L,), jnp.int32)
  cols_lo = lanes + 1
  cols_hi = lanes + (L + 1)
  feat = feat_ref.at[j]

  def group(g, _):
    r0 = g * L
    ids = idx_ref[pl.ds(j * CHUNK + r0, L)].astype(jnp.float32)
    plsc.store_scatter(feat, [r0 + lanes, zeros], ids)
    for k in range(L):
      r = r0 + k
      rsplat = jnp.full((L,), 0, jnp.int32) + r
      lo = rows_ref[j, r, pl.ds(0, L)]
      hi = rows_ref[j, r, pl.ds(L, L)]
      plsc.store_scatter(feat, [rsplat, cols_lo], lo)
      plsc.store_scatter(feat, [rsplat, cols_hi], hi)
    return 0

  lax.fori_loop(0, CHUNK // L, group, 0, unroll=False)


def _body(uid2, iid2, ut, it, uout, iout,
          uidx, iidx, urows, irows, feat, usem, isem):
  wid = lax.axis_index("s") * NC + lax.axis_index("c")
  base = wid * BW

  pltpu.sync_copy(uid2.at[wid], uidx)
  pltpu.sync_copy(iid2.at[wid], iidx)

  ucopies = [
      pltpu.async_copy(ut.at[uidx.at[pl.ds(j * CHUNK, CHUNK)]],
                       urows.at[j], usem)
      for j in range(NCH)
  ]
  icopies = [
      pltpu.async_copy(it.at[iidx.at[pl.ds(j * CHUNK, CHUNK)]],
                       irows.at[j], isem)
      for j in range(NCH)
  ]

  for c in ucopies:
    c.wait()
  for j in range(NCH):
    _assemble_chunk(uidx, urows, feat, j)
    pltpu.sync_copy(feat.at[j],
                    uout.at[pl.ds(base + j * CHUNK, CHUNK)])

  for c in icopies:
    c.wait()
  for j in range(NCH):
    _assemble_chunk(iidx, irows, feat, j)
    pltpu.sync_copy(feat.at[j],
                    iout.at[pl.ds(base + j * CHUNK, CHUNK)])


_sc_call = functools.partial(
    pl.kernel,
    out_type=[
        jax.ShapeDtypeStruct((B, OUT_D), jnp.float32),
        jax.ShapeDtypeStruct((B, OUT_D), jnp.float32),
    ],
    mesh=_mesh,
    scratch_types=[
        pltpu.VMEM((BW,), jnp.int32),
        pltpu.VMEM((BW,), jnp.int32),
        pltpu.VMEM((NCH, CHUNK, EMB), jnp.float32),
        pltpu.VMEM((NCH, CHUNK, EMB), jnp.float32),
        pltpu.VMEM((NCH, CHUNK, OUT_D), jnp.float32),
        pltpu.SemaphoreType.DMA,
        pltpu.SemaphoreType.DMA,
    ],
    compiler_params=pltpu.CompilerParams(use_tc_tiling_on_sc=False,
                                         needs_layout_passes=False),
)(_body)


@jax.jit
def kernel(user_id, item_id, user_table, item_table):
  uid2 = user_id.reshape(NW, BW).astype(jnp.int32)
  iid2 = item_id.reshape(NW, BW).astype(jnp.int32)
  return tuple(_sc_call(uid2, iid2, user_table, item_table))


# speedup vs baseline: 1.0084x; 1.0084x over previous
import functools

import jax
import jax.numpy as jnp
from jax import lax
from jax.experimental import pallas as pl
from jax.experimental.pallas import tpu as pltpu
from jax.experimental.pallas import tpu_sc as plsc

B = 16384
EMB = 32
OUT_D = EMB + 1
VOCAB4 = 250000
NC, NS, L = 2, 16, 16
NW = NC * NS
BW = B // NW
CHUNK = 128
NCH = BW // CHUNK
NSLOT = 2

_mesh = plsc.VectorSubcoreMesh(core_axis_name="c", subcore_axis_name="s")


def _prep_rowidx(idx_ref, rowidx_ref):
  lanes = lax.iota(jnp.int32, L)

  def group(g, _):
    sv = g * L + lanes
    ids = plsc.load_gather(idx_ref, [sv])
    plsc.store_scatter(rowidx_ref, [sv], jax.lax.shift_right_logical(ids, 2))
    return 0

  lax.fori_loop(0, BW // L, group, 0, unroll=False)


def _extract_chunk(idx_ref, rows_ref, feat_ref, j):
  lanes = lax.iota(jnp.int32, L)
  zeros = jnp.zeros((L,), jnp.int32)

  def group(g, _):
    sv = j * CHUNK + g * L + lanes
    lid = g * L + lanes
    ids = plsc.load_gather(idx_ref, [sv])
    plsc.store_scatter(feat_ref, [zeros, sv], ids.astype(jnp.float32))
    colbase = (ids & 3) * EMB
    for r in range(EMB):
      vals = plsc.load_gather(rows_ref, [lid, colbase + r])
      plsc.store_scatter(feat_ref, [zeros + (1 + r), sv], vals)
    return 0

  lax.fori_loop(0, CHUNK // L, group, 0, unroll=False)


def _body(uids, iids, ut4, it4, uoutT, ioutT,
          uidx, iidx, urowidx, irowidx, urows, irows, ufeat, ifeat,
          usems, isems):
  wid = lax.axis_index("s") * NC + lax.axis_index("c")
  base = wid * BW

  pltpu.sync_copy(uids.at[pl.ds(base, BW)], uidx)
  pltpu.sync_copy(iids.at[pl.ds(base, BW)], iidx)
  _prep_rowidx(uidx, urowidx)
  _prep_rowidx(iidx, irowidx)

  def fire(tbl, rowidx, rows, sems, j):
    return pltpu.async_copy(
        tbl.at[rowidx.at[pl.ds(j * CHUNK, CHUNK)]],
        rows.at[j % NSLOT], sems.at[j % NSLOT])

  ucopies = [fire(ut4, urowidx, urows, usems, j) for j in range(NSLOT)]
  icopies = [fire(it4, irowidx, irows, isems, j) for j in range(NSLOT)]

  for j in range(NCH):
    ucopies[j].wait()
    _extract_chunk(uidx, urows.at[j % NSLOT], ufeat, j)
    if j + NSLOT < NCH:
      ucopies.append(fire(ut4, urowidx, urows, usems, j + NSLOT))
  pltpu.sync_copy(ufeat, uoutT.at[:, pl.ds(base, BW)])

  for j in range(NCH):
    icopies[j].wait()
    _extract_chunk(iidx, irows.at[j % NSLOT], ifeat, j)
    if j + NSLOT < NCH:
      icopies.append(fire(it4, irowidx, irows, isems, j + NSLOT))
  pltpu.sync_copy(ifeat, ioutT.at[:, pl.ds(base, BW)])


_sc_call = functools.partial(
    pl.kernel,
    out_type=[
        jax.ShapeDtypeStruct((OUT_D, B), jnp.float32),
        jax.ShapeDtypeStruct((OUT_D, B), jnp.float32),
    ],
    mesh=_mesh,
    scratch_types=[
        pltpu.VMEM((BW,), jnp.int32),
        pltpu.VMEM((BW,), jnp.int32),
        pltpu.VMEM((BW,), jnp.int32),
        pltpu.VMEM((BW,), jnp.int32),
        pltpu.VMEM((NSLOT, CHUNK, 128), jnp.float32),
        pltpu.VMEM((NSLOT, CHUNK, 128), jnp.float32),
        pltpu.VMEM((OUT_D, BW), jnp.float32),
        pltpu.VMEM((OUT_D, BW), jnp.float32),
        pltpu.SemaphoreType.DMA((NSLOT,)),
        pltpu.SemaphoreType.DMA((NSLOT,)),
    ],
    compiler_params=pltpu.CompilerParams(needs_layout_passes=False),
)(_body)


@jax.jit
def kernel(user_id, item_id, user_table, item_table):
  uids = user_id.reshape(B).astype(jnp.int32)
  iids = item_id.reshape(B).astype(jnp.int32)
  ut4 = user_table.reshape(VOCAB4, 128)
  it4 = item_table.reshape(VOCAB4, 128)
  uT, iT = _sc_call(uids, iids, ut4, it4)
  return uT.T, iT.T
